# baseline (speedup 1.0000x reference)
import functools

import jax
import jax.numpy as jnp
from jax import lax
from jax.experimental import pallas as pl
from jax.experimental.pallas import tpu as pltpu
from jax.experimental.pallas import tpu_sc as plsc

NCLS = 18048
NFEAT = 256
NBATCH = 4096

NW = 32
BPW = NBATCH // NW
RCHUNK = 128
NCHUNK = NCLS // RCHUNK
KMAX = -(-NCHUNK // NW)

N_BLK = 384
MM_STEPS = NCLS // N_BLK
U_STEPS = (NBATCH + N_BLK - 1) // N_BLK
UM_ROWS = NCLS + NBATCH


def _sc_mesh():
    return plsc.VectorSubcoreMesh(core_axis_name="c", subcore_axis_name="s")


def _wid():
    return lax.axis_index("s") * 2 + lax.axis_index("c")


def _gather_body(mem_hbm, tgt_hbm, out_hbm, idx_v, rows_v, sem):
    w = _wid()
    base = w * BPW
    pltpu.sync_copy(tgt_hbm.at[pl.ds(base, BPW)], idx_v)
    pltpu.async_copy(mem_hbm.at[idx_v], rows_v, sem).wait()
    pltpu.sync_copy(rows_v, out_hbm.at[pl.ds(base, BPW)])


def _sc_gather(mem, targets):
    k = pl.kernel(
        _gather_body,
        out_type=jax.ShapeDtypeStruct((NBATCH, NFEAT), jnp.float32),
        mesh=_sc_mesh(),
        scratch_types=[
            pltpu.VMEM((BPW,), jnp.int32),
            pltpu.VMEM((BPW, NFEAT), jnp.float32),
            pltpu.SemaphoreType.DMA,
        ],
        name="sc_gather_rows",
    )
    return k(mem, targets)


def _val_body(t_col_ref, t_row_ref, val_ref):
    t_row = t_row_ref[...]

    def blk(b, _):
        tb = t_col_ref[pl.ds(b * 512, 512), :]
        eq = tb == t_row
        jidx = lax.broadcasted_iota(jnp.int32, (512, NBATCH), 1)
        lastocc = jnp.max(jnp.where(eq, jidx, -1), axis=1, keepdims=True)
        i_col = (lax.broadcasted_iota(jnp.int32, (512, 1), 0) + b * 512)
        iswin = lastocc == i_col
        val_ref[pl.ds(b * 512, 512), :] = jnp.where(
            iswin, NCLS + i_col - tb, 0)
        return 0

    lax.fori_loop(0, NBATCH // 512, blk, 0)


def _tc_val(t_col, t_row):
    return pl.pallas_call(
        _val_body,
        out_shape=jax.ShapeDtypeStruct((NBATCH, 1), jnp.int32),
        name="tc_winner_val",
    )(t_col, t_row)


def _mm_body(alpha_ref, x_mm_ref, mem_ref, g_ref, x_u_ref, logits_ref, um_ref):
    s = pl.program_id(0)

    @pl.when(s < MM_STEPS)
    def _matmul_and_copy():
        m = mem_ref[...]
        logits_ref[...] = lax.dot_general(
            x_mm_ref[...], m, (((1,), (1,)), ((), ())),
            preferred_element_type=jnp.float32,
            precision=lax.Precision.HIGHEST)
        um_ref[...] = m

    @pl.when(s >= MM_STEPS)
    def _update_rows():
        a = alpha_ref[0, 0]
        u = a * g_ref[...] + (1.0 - a) * x_u_ref[...]
        n = jnp.sqrt(jnp.sum(u * u, axis=1, keepdims=True))
        um_ref[...] = u / (n + 1e-12)


def _tc_matmul_um(alpha, inputs, mem, g):
    grid = (MM_STEPS + U_STEPS,)
    return pl.pallas_call(
        _mm_body,
        grid=grid,
        in_specs=[
            pl.BlockSpec(memory_space=pltpu.SMEM),
            pl.BlockSpec((NBATCH, NFEAT), lambda s: (0, 0)),
            pl.BlockSpec((N_BLK, NFEAT), lambda s: (jnp.minimum(s, MM_STEPS - 1), 0)),
            pl.BlockSpec((N_BLK, NFEAT), lambda s: (jnp.maximum(s - MM_STEPS, 0), 0)),
            pl.BlockSpec((N_BLK, NFEAT), lambda s: (jnp.maximum(s - MM_STEPS, 0), 0)),
        ],
        out_specs=[
            pl.BlockSpec((NBATCH, N_BLK), lambda s: (0, jnp.minimum(s, MM_STEPS - 1))),
            pl.BlockSpec((N_BLK, NFEAT), lambda s: (s, 0)),
        ],
        out_shape=[
            jax.ShapeDtypeStruct((NBATCH, NCLS), jnp.float32),
            jax.ShapeDtypeStruct((UM_ROWS, NFEAT), jnp.float32),
        ],
        name="tc_matmul_um",
    )(alpha, inputs, mem, g, inputs)


def _build_body(um_hbm, tgt_hbm, val_hbm, iota_hbm, out_hbm,
                srcidx_v, t_v, val_v, rows_v, sem):
    w = _wid()

    pltpu.sync_copy(iota_hbm, srcidx_v)
    pltpu.sync_copy(tgt_hbm, t_v)
    pltpu.sync_copy(val_hbm, val_v)

    def upd(i, _):
        idx = t_v[pl.ds(i * 16, 16)]
        v = val_v[pl.ds(i * 16, 16)]
        plsc.addupdate_scatter(srcidx_v, [idx], v, mask=v != 0)
        return 0

    lax.fori_loop(0, NBATCH // 16, upd, 0)

    for k in range(KMAX):
        c = w + k * NW

        @pl.when(c < NCHUNK)
        def _do_chunk():
            base = c * RCHUNK
            idx = srcidx_v.at[pl.ds(base, RCHUNK)]
            pltpu.async_copy(um_hbm.at[idx], rows_v, sem).wait()
            pltpu.sync_copy(rows_v, out_hbm.at[pl.ds(base, RCHUNK)])


def _sc_build(um, targets, val, iota0):
    k = pl.kernel(
        _build_body,
        out_type=jax.ShapeDtypeStruct((NCLS, NFEAT), jnp.float32),
        mesh=_sc_mesh(),
        scratch_types=[
            pltpu.VMEM((NCLS,), jnp.int32),
            pltpu.VMEM((NBATCH,), jnp.int32),
            pltpu.VMEM((NBATCH,), jnp.int32),
            pltpu.VMEM((RCHUNK, NFEAT), jnp.float32),
            pltpu.SemaphoreType.DMA,
        ],
        compiler_params=pltpu.CompilerParams(needs_layout_passes=False),
        name="sc_build_newmem",
    )
    return k(um, targets, val, iota0)


def kernel(inputs, targets, mem, epoch):
    t32 = targets.astype(jnp.int32)
    alpha = jnp.asarray(0.5 * epoch / 60.0, jnp.float32).reshape(1, 1)
    iota0 = lax.iota(jnp.int32, NCLS)

    g = _sc_gather(mem, t32)
    val = _tc_val(t32.reshape(NBATCH, 1), t32.reshape(1, NBATCH))
    logits, um = _tc_matmul_um(alpha, inputs, mem, g)
    new_mem = _sc_build(um, t32, val.reshape(NBATCH), iota0)
    return logits, new_mem


# speedup vs baseline: 2.2421x; 2.2421x over previous
import functools

import jax
import jax.numpy as jnp
from jax import lax
from jax.experimental import pallas as pl
from jax.experimental.pallas import tpu as pltpu
from jax.experimental.pallas import tpu_sc as plsc

NCLS = 18048
NFEAT = 256
NBATCH = 4096

NW = 32
BPW = NBATCH // NW
RCHUNK = 128
NCHUNK = NCLS // RCHUNK
KMAX = -(-NCHUNK // NW)

N_BLK = 384
MM_STEPS = NCLS // N_BLK
U_STEPS = (NBATCH + N_BLK - 1) // N_BLK
UM_ROWS = NCLS + NBATCH


def _sc_mesh():
    return plsc.VectorSubcoreMesh(core_axis_name="c", subcore_axis_name="s")


def _wid():
    return lax.axis_index("s") * 2 + lax.axis_index("c")


def _gather_body(mem_hbm, tgt_hbm, out_hbm, idx_v, rows_v, sem):
    w = _wid()
    base = w * BPW
    pltpu.sync_copy(tgt_hbm.at[pl.ds(base, BPW)], idx_v)
    pltpu.async_copy(mem_hbm.at[idx_v], rows_v, sem).wait()
    pltpu.sync_copy(rows_v, out_hbm.at[pl.ds(base, BPW)])


def _sc_gather(mem, targets):
    k = pl.kernel(
        _gather_body,
        out_type=jax.ShapeDtypeStruct((NBATCH, NFEAT), jnp.float32),
        mesh=_sc_mesh(),
        scratch_types=[
            pltpu.VMEM((BPW,), jnp.int32),
            pltpu.VMEM((BPW, NFEAT), jnp.float32),
            pltpu.SemaphoreType.DMA,
        ],
        name="sc_gather_rows",
    )
    return k(mem, targets)


def _val_body(t_col_ref, t_row_ref, val_ref):
    t_row = t_row_ref[...]

    def blk(b, _):
        tb = t_col_ref[pl.ds(b * 512, 512), :]
        eq = tb == t_row
        jidx = lax.broadcasted_iota(jnp.int32, (512, NBATCH), 1)
        lastocc = jnp.max(jnp.where(eq, jidx, -1), axis=1, keepdims=True)
        i_col = (lax.broadcasted_iota(jnp.int32, (512, 1), 0) + b * 512)
        iswin = lastocc == i_col
        val_ref[pl.ds(b * 512, 512), :] = jnp.where(
            iswin, NCLS + i_col - tb, 0)
        return 0

    lax.fori_loop(0, NBATCH // 512, blk, 0)


def _tc_val(t_col, t_row):
    return pl.pallas_call(
        _val_body,
        out_shape=jax.ShapeDtypeStruct((NBATCH, 1), jnp.int32),
        name="tc_winner_val",
    )(t_col, t_row)


def _mm_body(alpha_ref, x_mm_ref, mem_ref, g_ref, x_u_ref, logits_ref, um_ref):
    s = pl.program_id(0)

    @pl.when(s < MM_STEPS)
    def _matmul_and_copy():
        m = mem_ref[...]
        logits_ref[...] = lax.dot_general(
            x_mm_ref[...], m, (((1,), (1,)), ((), ())),
            preferred_element_type=jnp.float32)
        um_ref[...] = m

    @pl.when(s >= MM_STEPS)
    def _update_rows():
        a = alpha_ref[0, 0]
        u = a * g_ref[...] + (1.0 - a) * x_u_ref[...]
        n = jnp.sqrt(jnp.sum(u * u, axis=1, keepdims=True))
        um_ref[...] = u / (n + 1e-12)


def _tc_matmul_um(alpha, inputs, mem, g):
    grid = (MM_STEPS + U_STEPS,)
    return pl.pallas_call(
        _mm_body,
        grid=grid,
        in_specs=[
            pl.BlockSpec(memory_space=pltpu.SMEM),
            pl.BlockSpec((NBATCH, NFEAT), lambda s: (0, 0)),
            pl.BlockSpec((N_BLK, NFEAT), lambda s: (jnp.minimum(s, MM_STEPS - 1), 0)),
            pl.BlockSpec((N_BLK, NFEAT), lambda s: (jnp.maximum(s - MM_STEPS, 0), 0)),
            pl.BlockSpec((N_BLK, NFEAT), lambda s: (jnp.maximum(s - MM_STEPS, 0), 0)),
        ],
        out_specs=[
            pl.BlockSpec((NBATCH, N_BLK), lambda s: (0, jnp.minimum(s, MM_STEPS - 1))),
            pl.BlockSpec((N_BLK, NFEAT), lambda s: (s, 0)),
        ],
        out_shape=[
            jax.ShapeDtypeStruct((NBATCH, NCLS), jnp.float32),
            jax.ShapeDtypeStruct((UM_ROWS, NFEAT), jnp.float32),
        ],
        name="tc_matmul_um",
    )(alpha, inputs, mem, g, inputs)


def _build_body(um_hbm, tgt_hbm, val_hbm, iota_hbm, out_hbm,
                srcidx_v, t_v, val_v, rows_v, sem):
    w = _wid()

    pltpu.sync_copy(iota_hbm, srcidx_v)
    pltpu.sync_copy(tgt_hbm, t_v)
    pltpu.sync_copy(val_hbm, val_v)

    def upd(i, _):
        idx = t_v[pl.ds(i * 16, 16)]
        v = val_v[pl.ds(i * 16, 16)]
        plsc.addupdate_scatter(srcidx_v, [idx], v, mask=v != 0)
        return 0

    lax.fori_loop(0, NBATCH // 16, upd, 0)

    for k in range(KMAX):
        c = w + k * NW

        @pl.when(c < NCHUNK)
        def _do_chunk():
            base = c * RCHUNK
            idx = srcidx_v.at[pl.ds(base, RCHUNK)]
            pltpu.async_copy(um_hbm.at[idx], rows_v, sem).wait()
            pltpu.sync_copy(rows_v, out_hbm.at[pl.ds(base, RCHUNK)])


def _sc_build(um, targets, val, iota0):
    k = pl.kernel(
        _build_body,
        out_type=jax.ShapeDtypeStruct((NCLS, NFEAT), jnp.float32),
        mesh=_sc_mesh(),
        scratch_types=[
            pltpu.VMEM((NCLS,), jnp.int32),
            pltpu.VMEM((NBATCH,), jnp.int32),
            pltpu.VMEM((NBATCH,), jnp.int32),
            pltpu.VMEM((RCHUNK, NFEAT), jnp.float32),
            pltpu.SemaphoreType.DMA,
        ],
        compiler_params=pltpu.CompilerParams(needs_layout_passes=False),
        name="sc_build_newmem",
    )
    return k(um, targets, val, iota0)


def kernel(inputs, targets, mem, epoch):
    t32 = targets.astype(jnp.int32)
    alpha = jnp.asarray(0.5 * epoch / 60.0, jnp.float32).reshape(1, 1)
    iota0 = lax.iota(jnp.int32, NCLS)

    g = _sc_gather(mem, t32)
    val = _tc_val(t32.reshape(NBATCH, 1), t32.reshape(1, NBATCH))
    logits, um = _tc_matmul_um(alpha, inputs, mem, g)
    new_mem = _sc_build(um, t32, val.reshape(NBATCH), iota0)
    return logits, new_mem


# speedup vs baseline: 2.4288x; 1.0833x over previous
import jax
import jax.numpy as jnp
from jax import lax
from jax.experimental import pallas as pl
from jax.experimental.pallas import tpu as pltpu
from jax.experimental.pallas import tpu_sc as plsc

NCLS = 18048
NFEAT = 256
NBATCH = 4096

NW = 32
BPW = NBATCH // NW

N_BLK = 384
MM_STEPS = NCLS // N_BLK
U_STEPS = (NBATCH + N_BLK - 1) // N_BLK
LO_BLK = 128
LO_STEPS = NBATCH // LO_BLK


def _sc_mesh():
    return plsc.VectorSubcoreMesh(core_axis_name="c", subcore_axis_name="s")


def _wid():
    return lax.axis_index("s") * 2 + lax.axis_index("c")


def _gather_body(mem_hbm, tgt_hbm, out_hbm, idx_v, rows_v, sem):
    w = _wid()
    base = w * BPW
    pltpu.sync_copy(tgt_hbm.at[pl.ds(base, BPW)], idx_v)
    pltpu.async_copy(mem_hbm.at[idx_v], rows_v, sem).wait()
    pltpu.sync_copy(rows_v, out_hbm.at[pl.ds(base, BPW)])


def _sc_gather(mem, targets):
    k = pl.kernel(
        _gather_body,
        out_type=jax.ShapeDtypeStruct((NBATCH, NFEAT), jnp.float32),
        mesh=_sc_mesh(),
        scratch_types=[
            pltpu.VMEM((BPW,), jnp.int32),
            pltpu.VMEM((BPW, NFEAT), jnp.float32),
            pltpu.SemaphoreType.DMA,
        ],
        name="sc_gather_rows",
    )
    return k(mem, targets)


def _mm_body(alpha_ref, x_mm_ref, mem_ref, g_ref, x_u_ref, t_col_ref,
             t_row_ref, logits_ref, u_ref, lo_ref):
    s = pl.program_id(0)

    @pl.when(s < MM_STEPS)
    def _matmul():
        logits_ref[...] = lax.dot_general(
            x_mm_ref[...], mem_ref[...], (((1,), (1,)), ((), ())),
            preferred_element_type=jnp.float32)

    @pl.when(s < LO_STEPS)
    def _lastocc_block():
        tb = t_col_ref[pl.ds(s * LO_BLK, LO_BLK), :]
        eq = tb == t_row_ref[...]
        jidx = lax.broadcasted_iota(jnp.int32, (LO_BLK, NBATCH), 1)
        lo_ref[pl.ds(s * LO_BLK, LO_BLK), :] = jnp.max(
            jnp.where(eq, jidx, -1), axis=1, keepdims=True)

    @pl.when(s >= MM_STEPS)
    def _update_rows():
        a = alpha_ref[0, 0]
        u = a * g_ref[...] + (1.0 - a) * x_u_ref[...]
        n = jnp.sqrt(jnp.sum(u * u, axis=1, keepdims=True))
        u_ref[...] = u / (n + 1e-12)


def _tc_main(alpha, inputs, mem, g, t_col, t_row):
    grid = (MM_STEPS + U_STEPS,)
    return pl.pallas_call(
        _mm_body,
        grid=grid,
        in_specs=[
            pl.BlockSpec(memory_space=pltpu.SMEM),
            pl.BlockSpec((NBATCH, NFEAT), lambda s: (0, 0)),
            pl.BlockSpec((N_BLK, NFEAT), lambda s: (jnp.minimum(s, MM_STEPS - 1), 0)),
            pl.BlockSpec((N_BLK, NFEAT), lambda s: (jnp.maximum(s - MM_STEPS, 0), 0)),
            pl.BlockSpec((N_BLK, NFEAT), lambda s: (jnp.maximum(s - MM_STEPS, 0), 0)),
            pl.BlockSpec((NBATCH, 1), lambda s: (0, 0)),
            pl.BlockSpec((1, NBATCH), lambda s: (0, 0)),
        ],
        out_specs=[
            pl.BlockSpec((NBATCH, N_BLK), lambda s: (0, jnp.minimum(s, MM_STEPS - 1))),
            pl.BlockSpec((N_BLK, NFEAT), lambda s: (jnp.maximum(s - MM_STEPS, 0), 0)),
            pl.BlockSpec((NBATCH, 1), lambda s: (0, 0)),
        ],
        out_shape=[
            jax.ShapeDtypeStruct((NBATCH, NCLS), jnp.float32),
            jax.ShapeDtypeStruct((NBATCH, NFEAT), jnp.float32),
            jax.ShapeDtypeStruct((NBATCH, 1), jnp.int32),
        ],
        name="tc_matmul_lo_u",
    )(alpha, inputs, mem, g, inputs, t_col, t_row)


def _scatterf_body(u_hbm, tgt_hbm, lo_hbm, nm_hbm, idx_v, lo_v, rows_v, sem):
    w = _wid()
    base = w * BPW
    pltpu.sync_copy(tgt_hbm.at[pl.ds(base, BPW)], idx_v)
    pltpu.sync_copy(lo_hbm.at[pl.ds(base, BPW)], lo_v)
    pltpu.async_copy(u_hbm.at[lo_v], rows_v, sem).wait()
    pltpu.async_copy(rows_v, nm_hbm.at[idx_v], sem).wait()


def _sc_scatterf(u, targets, lastocc, nm_ref):
    k = pl.kernel(
        _scatterf_body,
        out_type=(),
        mesh=_sc_mesh(),
        scratch_types=[
            pltpu.VMEM((BPW,), jnp.int32),
            pltpu.VMEM((BPW,), jnp.int32),
            pltpu.VMEM((BPW, NFEAT), jnp.float32),
            pltpu.SemaphoreType.DMA,
        ],
        name="sc_scatter_winners",
    )
    k(u, targets, lastocc, nm_ref)


def kernel(inputs, targets, mem, epoch):
    t32 = targets.astype(jnp.int32)
    alpha = jnp.asarray(0.5 * epoch / 60.0, jnp.float32).reshape(1, 1)

    g = _sc_gather(mem, t32)
    logits, u, lastocc = _tc_main(
        alpha, inputs, mem, g,
        t32.reshape(NBATCH, 1), t32.reshape(1, NBATCH))
    nm_ref = jax.new_ref(mem)
    _sc_scatterf(u, t32, lastocc.reshape(NBATCH), nm_ref)
    return logits, nm_ref[...]


# speedup vs baseline: 2.5485x; 1.0493x over previous
import jax
import jax.numpy as jnp
from jax import lax
from jax.experimental import pallas as pl
from jax.experimental.pallas import tpu as pltpu
from jax.experimental.pallas import tpu_sc as plsc

NCLS = 18048
NFEAT = 256
NBATCH = 4096

NW = 32
BPW = NBATCH // NW

N_BLK = 384
MM_STEPS = NCLS // N_BLK
U_STEPS = (NBATCH + N_BLK - 1) // N_BLK
LO_BLK = 128
LO_STEPS = NBATCH // LO_BLK


def _sc_mesh():
    return plsc.VectorSubcoreMesh(core_axis_name="c", subcore_axis_name="s")


def _wid():
    return lax.axis_index("s") * 2 + lax.axis_index("c")


def _gather_body(mem_hbm, tgt_hbm, out_hbm, idx_v, rows_v, sem):
    w = _wid()
    base = w * BPW
    pltpu.sync_copy(tgt_hbm.at[pl.ds(base, BPW)], idx_v)
    pltpu.async_copy(mem_hbm.at[idx_v], rows_v, sem).wait()
    pltpu.sync_copy(rows_v, out_hbm.at[pl.ds(base, BPW)])


def _sc_gather(mem, targets):
    k = pl.kernel(
        _gather_body,
        out_type=jax.ShapeDtypeStruct((NBATCH, NFEAT), jnp.float32),
        mesh=_sc_mesh(),
        scratch_types=[
            pltpu.VMEM((BPW,), jnp.int32),
            pltpu.VMEM((BPW, NFEAT), jnp.float32),
            pltpu.SemaphoreType.DMA,
        ],
        name="sc_gather_rows",
    )
    return k(mem, targets)


def _mm_body(x_mm_ref, mem_ref, t_col_ref, t_row_ref, logits_ref, lo_ref):
    s = pl.program_id(0)

    logits_ref[...] = lax.dot_general(
        x_mm_ref[...], mem_ref[...], (((1,), (1,)), ((), ())),
        preferred_element_type=jnp.float32)

    b = lax.rem(s, LO_STEPS)
    tb = t_col_ref[pl.ds(b * LO_BLK, LO_BLK), :]
    eq = tb == t_row_ref[...]
    jidx = lax.broadcasted_iota(jnp.int32, (LO_BLK, NBATCH), 1)
    lo_ref[pl.ds(b * LO_BLK, LO_BLK), :] = jnp.max(
        jnp.where(eq, jidx, -1), axis=1, keepdims=True)


def _tc_main(inputs, mem, t_col, t_row):
    return pl.pallas_call(
        _mm_body,
        grid=(MM_STEPS,),
        in_specs=[
            pl.BlockSpec((NBATCH, NFEAT), lambda s: (0, 0)),
            pl.BlockSpec((N_BLK, NFEAT), lambda s: (s, 0)),
            pl.BlockSpec((NBATCH, 1), lambda s: (0, 0)),
            pl.BlockSpec((1, NBATCH), lambda s: (0, 0)),
        ],
        out_specs=[
            pl.BlockSpec((NBATCH, N_BLK), lambda s: (0, s)),
            pl.BlockSpec((NBATCH, 1), lambda s: (0, 0)),
        ],
        out_shape=[
            jax.ShapeDtypeStruct((NBATCH, NCLS), jnp.float32),
            jax.ShapeDtypeStruct((NBATCH, 1), jnp.int32),
        ],
        name="tc_matmul_lo",
    )(inputs, mem, t_col, t_row)


def _upd_body(alpha_ref, g_ref, x_ref, u_ref):
    a = alpha_ref[0, 0]
    u = a * g_ref[...] + (1.0 - a) * x_ref[...]
    n = jnp.sqrt(jnp.sum(u * u, axis=1, keepdims=True))
    u_ref[...] = u / (n + 1e-12)


def _tc_update(alpha, g, inputs):
    return pl.pallas_call(
        _upd_body,
        in_specs=[
            pl.BlockSpec(memory_space=pltpu.SMEM),
            pl.BlockSpec((NBATCH, NFEAT), lambda: (0, 0)),
            pl.BlockSpec((NBATCH, NFEAT), lambda: (0, 0)),
        ],
        out_specs=pl.BlockSpec((NBATCH, NFEAT), lambda: (0, 0)),
        out_shape=jax.ShapeDtypeStruct((NBATCH, NFEAT), jnp.float32),
        name="tc_update_u",
    )(alpha, g, inputs)


def _scatterf_body(u_hbm, tgt_hbm, lo_hbm, nm_hbm, idx_v, lo_v, rows_v, sem):
    w = _wid()
    base = w * BPW
    pltpu.sync_copy(tgt_hbm.at[pl.ds(base, BPW)], idx_v)
    pltpu.sync_copy(lo_hbm.at[pl.ds(base, BPW)], lo_v)
    pltpu.async_copy(u_hbm.at[lo_v], rows_v, sem).wait()
    pltpu.async_copy(rows_v, nm_hbm.at[idx_v], sem).wait()


def _sc_scatterf(u, targets, lastocc, nm_ref):
    k = pl.kernel(
        _scatterf_body,
        out_type=(),
        mesh=_sc_mesh(),
        scratch_types=[
            pltpu.VMEM((BPW,), jnp.int32),
            pltpu.VMEM((BPW,), jnp.int32),
            pltpu.VMEM((BPW, NFEAT), jnp.float32),
            pltpu.SemaphoreType.DMA,
        ],
        name="sc_scatter_winners",
    )
    k(u, targets, lastocc, nm_ref)


def kernel(inputs, targets, mem, epoch):
    t32 = targets.astype(jnp.int32)
    alpha = jnp.asarray(0.5 * epoch / 60.0, jnp.float32).reshape(1, 1)

    g = _sc_gather(mem, t32)
    logits, lastocc = _tc_main(
        inputs, mem, t32.reshape(NBATCH, 1), t32.reshape(1, NBATCH))
    u = _tc_update(alpha, g, inputs)
    nm_ref = jax.new_ref(mem)
    _sc_scatterf(u, t32, lastocc.reshape(NBATCH), nm_ref)
    return logits, nm_ref[...]
